# Initial kernel scaffold; baseline (speedup 1.0000x reference)
import jax
import jax.numpy as jnp
from jax.experimental import pallas as pl


def kernel(var_feat, con_feat, edge_index, vW1, vb1, vW2, vb2, cW1, cb1, cW2, cb2, lvW, lvb, lcW, lcb):
    raise NotImplementedError("write your pallas kernel here")


# speedup vs baseline: 7.0420x; 7.0420x over previous
import functools

import jax
import jax.numpy as jnp
from jax import lax
from jax.experimental import pallas as pl
from jax.experimental.pallas import tpu as pltpu
from jax.experimental.pallas import tpu_sc as plsc

NV = 50000
NE = 800000
DIN = 32
DH = 64
HALF = 32
NLAYER = 3

NTILE = 16
CH = 128
GRP = 8
EPT = NE // NTILE
CPT = 392
EPT_PAD = CPT * CH
NGRP = CPT // GRP
SROWS = NTILE * CPT
RPT = NV // NTILE
ZROWS = 125
ACC_ROWS = NV + 8

_sc_cache = {}


def _sc_agg(table, gidx, sidx):
    if "agg" not in _sc_cache:
        mesh = plsc.VectorSubcoreMesh(core_axis_name="c", subcore_axis_name="s")
        _sc_cache["agg"] = functools.partial(
            pl.kernel,
            mesh=mesh,
            compiler_params=pltpu.CompilerParams(use_tc_tiling_on_sc=False),
            out_type=jax.ShapeDtypeStruct((2 * NV, HALF), jnp.float32),
            scratch_types=[
                pltpu.VMEM((GRP, CH), jnp.int32),
                pltpu.VMEM((GRP, CH), jnp.int32),
                pltpu.VMEM((CH, HALF), jnp.float32),
                pltpu.VMEM((CH, HALF), jnp.float32),
                pltpu.VMEM((ZROWS, HALF), jnp.float32),
                pltpu.VMEM_SHARED((ACC_ROWS, HALF), jnp.float32),
                pltpu.SemaphoreType.DMA,
                pltpu.SemaphoreType.DMA,
            ],
        )(_sc_agg_body)
    return _sc_cache["agg"](table, gidx, sidx)


def _sc_agg_body(table_h, gidx_h, sidx_h, out_h, si_v, gi_v, rows0, rows1, zb,
                 acc, sem0, sem1):
    cid = lax.axis_index("c")
    sid = lax.axis_index("s")

    def zfill(i, carry):
        zb[i, pl.ds(0, 16)] = jnp.zeros((16,), jnp.float32)
        zb[i, pl.ds(16, 16)] = jnp.zeros((16,), jnp.float32)
        return carry

    lax.fori_loop(0, ZROWS, zfill, 0)

    rowbase = sid * RPT

    def zcopy(i, carry):
        pltpu.sync_copy(zb, acc.at[pl.ds(rowbase + i * ZROWS, ZROWS)])
        return carry

    lax.fori_loop(0, RPT // ZROWS, zcopy, 0)
    plsc.subcore_barrier()

    tbl = table_h.at[pl.ds(pl.multiple_of(cid * NV, 8), NV)]
    row0 = sid * CPT
    bufs = (rows0, rows1)
    sems = (sem0, sem1)

    def group(g, carry):
        pltpu.sync_copy(sidx_h.at[pl.ds(row0 + g * GRP, GRP)], si_v)
        pltpu.sync_copy(gidx_h.at[pl.ds(row0 + g * GRP, GRP)], gi_v)
        hs = [None] * GRP
        hs[0] = pltpu.async_copy(tbl.at[gi_v.at[0]], bufs[0], sems[0])
        for j in range(GRP):
            if j + 1 < GRP:
                hs[j + 1] = pltpu.async_copy(
                    tbl.at[gi_v.at[j + 1]], bufs[(j + 1) % 2],
                    sems[(j + 1) % 2])
            hs[j].wait()
            pltpu.sync_copy(bufs[j % 2], acc.at[si_v.at[j]], add=True)
        return carry

    lax.fori_loop(0, NGRP, group, 0)
    plsc.subcore_barrier()

    @pl.when(sid == 0)
    def _():
        pltpu.sync_copy(
            acc.at[pl.ds(0, NV)],
            out_h.at[pl.ds(pl.multiple_of(cid * NV, 8), NV)])


def _sc_deg(idx):
    if "deg" not in _sc_cache:
        mesh = plsc.VectorSubcoreMesh(core_axis_name="c", subcore_axis_name="s")
        _sc_cache["deg"] = functools.partial(
            pl.kernel,
            mesh=mesh,
            compiler_params=pltpu.CompilerParams(use_tc_tiling_on_sc=False),
            out_type=jax.ShapeDtypeStruct((2 * NV, 16), jnp.float32),
            scratch_types=[
                pltpu.VMEM((GRP, CH), jnp.int32),
                pltpu.VMEM((CH, 16), jnp.float32),
                pltpu.VMEM((ZROWS, 16), jnp.float32),
                pltpu.VMEM_SHARED((ACC_ROWS, 16), jnp.float32),
            ],
        )(_sc_deg_body)
    return _sc_cache["deg"](idx)


def _sc_deg_body(idx_h, out_h, si_v, ones_v, zb, acc):
    cid = lax.axis_index("c")
    sid = lax.axis_index("s")

    def zfill(i, carry):
        zb[i, pl.ds(0, 16)] = jnp.zeros((16,), jnp.float32)
        return carry

    lax.fori_loop(0, ZROWS, zfill, 0)

    def ofill(i, carry):
        ones_v[i, pl.ds(0, 16)] = jnp.full((16,), 1.0, jnp.float32)
        return carry

    lax.fori_loop(0, CH, ofill, 0)

    rowbase = sid * RPT

    def zcopy(i, carry):
        pltpu.sync_copy(zb, acc.at[pl.ds(rowbase + i * ZROWS, ZROWS)])
        return carry

    lax.fori_loop(0, RPT // ZROWS, zcopy, 0)
    plsc.subcore_barrier()

    row0 = cid * SROWS + sid * CPT

    def group(g, carry):
        pltpu.sync_copy(
            idx_h.at[pl.ds(pl.multiple_of(row0, 8) + g * GRP, GRP)], si_v)
        for j in range(GRP):
            pltpu.sync_copy(ones_v, acc.at[si_v.at[j]], add=True)
        return carry

    lax.fori_loop(0, NGRP, group, 0)
    plsc.subcore_barrier()

    @pl.when(sid == 0)
    def _():
        pltpu.sync_copy(
            acc.at[pl.ds(0, NV)],
            out_h.at[pl.ds(pl.multiple_of(cid * NV, 8), NV)])


RB = 2000
NB = NV // RB


def _mlp_body(x_ref, w1_ref, b1_ref, w2_ref, b2_ref, o_ref):
    h = jnp.dot(x_ref[...], w1_ref[...], preferred_element_type=jnp.float32)
    h = jnp.maximum(h + b1_ref[...], 0.0)
    y = jnp.dot(h, w2_ref[...], preferred_element_type=jnp.float32)
    y = y + b2_ref[...]
    o_ref[0] = y[:, :HALF]
    o_ref[1] = y[:, HALF:]


def _mlp(x, w1, b1, w2, b2):
    return pl.pallas_call(
        _mlp_body,
        grid=(NB,),
        in_specs=[
            pl.BlockSpec((RB, DIN), lambda i: (i, 0)),
            pl.BlockSpec((DIN, DH), lambda i: (0, 0)),
            pl.BlockSpec((1, DH), lambda i: (0, 0)),
            pl.BlockSpec((DH, DH), lambda i: (0, 0)),
            pl.BlockSpec((1, DH), lambda i: (0, 0)),
        ],
        out_specs=pl.BlockSpec((2, RB, HALF), lambda i: (0, i, 0)),
        out_shape=jax.ShapeDtypeStruct((2, NV, HALF), jnp.float32),
    )(x, w1, b1.reshape(1, DH), w2, b2.reshape(1, DH))


def _update_body(split_out, x_ref, w_ref, b_ref, a_ref, d_ref, o_ref):
    x = jnp.concatenate([x_ref[0], x_ref[1]], axis=1)
    a = jnp.concatenate([a_ref[0], a_ref[1]], axis=1)
    inv = 1.0 / jnp.maximum(d_ref[:, 0:1], 1.0)
    y = jnp.dot(x, w_ref[...], preferred_element_type=jnp.float32)
    y = jnp.maximum(y + b_ref[...] + a * inv, 0.0)
    if split_out:
        o_ref[0] = y[:, :HALF]
        o_ref[1] = y[:, HALF:]
    else:
        o_ref[...] = y


def _update(x2, w, b, agg2, deg, split_out):
    if split_out:
        out_spec = pl.BlockSpec((2, RB, HALF), lambda i: (0, i, 0))
        out_shape = jax.ShapeDtypeStruct((2, NV, HALF), jnp.float32)
    else:
        out_spec = pl.BlockSpec((RB, DH), lambda i: (i, 0))
        out_shape = jax.ShapeDtypeStruct((NV, DH), jnp.float32)
    return pl.pallas_call(
        functools.partial(_update_body, split_out),
        grid=(NB,),
        in_specs=[
            pl.BlockSpec((2, RB, HALF), lambda i: (0, i, 0)),
            pl.BlockSpec((DH, DH), lambda i: (0, 0)),
            pl.BlockSpec((1, DH), lambda i: (0, 0)),
            pl.BlockSpec((2, RB, HALF), lambda i: (0, i, 0)),
            pl.BlockSpec((RB, 16), lambda i: (i, 0)),
        ],
        out_specs=out_spec,
        out_shape=out_shape,
    )(x2, w, b.reshape(1, DH), agg2, deg)


def _pad_chunks(idx16, pad_value):
    pad = jnp.full((NTILE, EPT_PAD - EPT), pad_value, jnp.int32)
    return jnp.concatenate([idx16, pad], axis=1).reshape(SROWS, CH)


def kernel(var_feat, con_feat, edge_index, vW1, vb1, vW2, vb2, cW1, cb1, cW2,
           cb2, lvW, lvb, lcW, lcb):
    src16 = edge_index[0].reshape(NTILE, EPT)
    dst16 = edge_index[1].reshape(NTILE, EPT)

    g_from_c = _pad_chunks(src16, 0)
    g_from_v = _pad_chunks(dst16, 0)
    s_to_v = _pad_chunks(dst16, NV)
    s_to_c = _pad_chunks(src16, NV)
    deg_idx = jnp.concatenate([s_to_v, s_to_c], axis=0)

    degs = _sc_deg(deg_idx)
    deg_v = degs[:NV]
    deg_c = degs[NV:]

    v2 = _mlp(var_feat, vW1, vb1, vW2, vb2)
    c2 = _mlp(con_feat, cW1, cb1, cW2, cb2)

    for l in range(NLAYER):
        agg_cv = _sc_agg(c2.reshape(2 * NV, HALF), g_from_c, s_to_v)
        agg_cv2 = agg_cv.reshape(2, NV, HALF)
        if l < NLAYER - 1:
            v2 = _update(v2, lvW[l], lvb[l], agg_cv2, deg_v, True)
            agg_vc = _sc_agg(v2.reshape(2 * NV, HALF), g_from_v, s_to_c)
            c2 = _update(c2, lcW[l], lcb[l], agg_vc.reshape(2, NV, HALF),
                         deg_c, True)
        else:
            v_out = _update(v2, lvW[l], lvb[l], agg_cv2, deg_v, False)
    return v_out


# speedup vs baseline: 8.9466x; 1.2705x over previous
import functools

import jax
import jax.numpy as jnp
from jax import lax
from jax.experimental import pallas as pl
from jax.experimental.pallas import tpu as pltpu
from jax.experimental.pallas import tpu_sc as plsc

NV = 50000
NE = 800000
DIN = 32
DH = 64
HALF = 32
NLAYER = 3

NTILE = 16
CH = 128
GRP = 8
NBUF = 4
EPT = NE // NTILE
CPT = 392
EPT_PAD = CPT * CH
NGRP = CPT // GRP
SROWS = NTILE * CPT
RPT = NV // NTILE
ZROWS = 125
ACC_ROWS = NV + 8

_sc_cache = {}


def _sc_agg(table, gidx, sidx):
    if "agg" not in _sc_cache:
        mesh = plsc.VectorSubcoreMesh(core_axis_name="c", subcore_axis_name="s")
        _sc_cache["agg"] = functools.partial(
            pl.kernel,
            mesh=mesh,
            compiler_params=pltpu.CompilerParams(use_tc_tiling_on_sc=False),
            out_type=jax.ShapeDtypeStruct((2 * NV, HALF), jnp.float32),
            scratch_types=[
                pltpu.VMEM((2, GRP, CH), jnp.int32),
                pltpu.VMEM((2, GRP, CH), jnp.int32),
                pltpu.VMEM((NBUF, CH, HALF), jnp.float32),
                pltpu.VMEM((ZROWS, HALF), jnp.float32),
                pltpu.VMEM_SHARED((ACC_ROWS, HALF), jnp.float32),
                pltpu.SemaphoreType.DMA,
                pltpu.SemaphoreType.DMA,
                pltpu.SemaphoreType.DMA,
                pltpu.SemaphoreType.DMA,
                pltpu.SemaphoreType.DMA,
            ],
        )(_sc_agg_body)
    return _sc_cache["agg"](table, gidx, sidx)


def _sc_agg_body(table_h, gidx_h, sidx_h, out_h, si2, gi2, rows, zb, acc,
                 sem0, sem1, sem2, sem3, isem):
    cid = lax.axis_index("c")
    sid = lax.axis_index("s")
    sems = (sem0, sem1, sem2, sem3)

    def zfill(i, carry):
        zb[i, pl.ds(0, 16)] = jnp.zeros((16,), jnp.float32)
        zb[i, pl.ds(16, 16)] = jnp.zeros((16,), jnp.float32)
        return carry

    lax.fori_loop(0, ZROWS, zfill, 0)

    rowbase = sid * RPT
    row0 = sid * CPT
    tbl = table_h.at[pl.ds(pl.multiple_of(cid * NV, 8), NV)]

    off0 = pl.multiple_of(row0, 8)
    pltpu.async_copy(sidx_h.at[pl.ds(off0, GRP)], si2.at[0], isem)
    pltpu.async_copy(gidx_h.at[pl.ds(off0, GRP)], gi2.at[0], isem)

    zhs = [
        pltpu.async_copy(zb, acc.at[pl.ds(rowbase + i * ZROWS, ZROWS)], sem0)
        for i in range(RPT // ZROWS)
    ]
    for h in zhs:
        h.wait()
    plsc.subcore_barrier()

    def ring_wait(sem):
        pltpu.make_async_copy(tbl.at[pl.ds(0, CH)], rows.at[0], sem).wait()

    def idx_wait():
        pltpu.make_async_copy(
            sidx_h.at[pl.ds(0, GRP)], si2.at[0], isem).wait()
        pltpu.make_async_copy(
            gidx_h.at[pl.ds(0, GRP)], gi2.at[0], isem).wait()

    def group(g, p, q, first):
        idx_wait()
        for j in range(GRP):
            ka = j % NBUF
            if not (first and j < 2):
                kb = (j - 2) % NBUF
                ring_wait(sems[kb])
                if j < 2:
                    splane, srow = q, GRP - 2 + j
                else:
                    splane, srow = p, j - 2
                pltpu.async_copy(
                    rows.at[kb], acc.at[si2.at[splane, srow]], sems[kb],
                    add=True)
            if not (first and j < NBUF):
                ring_wait(sems[ka])
            pltpu.async_copy(tbl.at[gi2.at[p, j]], rows.at[ka], sems[ka])
            if j == 3:
                off = pl.multiple_of(row0 + (g + 1) * GRP, 8)
                pltpu.async_copy(sidx_h.at[pl.ds(off, GRP)], si2.at[q], isem)
                pltpu.async_copy(gidx_h.at[pl.ds(off, GRP)], gi2.at[q], isem)

    group(0, 0, 1, True)

    def body(g, carry):
        p = lax.rem(g, 2)
        group(g, p, 1 - p, False)
        return carry

    lax.fori_loop(1, NGRP, body, 0)

    for j in (GRP - 2, GRP - 1):
        k = j % NBUF
        ring_wait(sems[k])
        pltpu.async_copy(rows.at[k], acc.at[si2.at[0, j]], sems[k], add=True)
    for k in range(NBUF):
        ring_wait(sems[k])
    idx_wait()
    plsc.subcore_barrier()

    @pl.when(sid == 0)
    def _():
        pltpu.sync_copy(
            acc.at[pl.ds(0, NV)],
            out_h.at[pl.ds(pl.multiple_of(cid * NV, 8), NV)])


def _sc_deg(idx):
    if "deg" not in _sc_cache:
        mesh = plsc.VectorSubcoreMesh(core_axis_name="c", subcore_axis_name="s")
        _sc_cache["deg"] = functools.partial(
            pl.kernel,
            mesh=mesh,
            compiler_params=pltpu.CompilerParams(use_tc_tiling_on_sc=False),
            out_type=jax.ShapeDtypeStruct((2 * NV, 16), jnp.float32),
            scratch_types=[
                pltpu.VMEM((2, GRP, CH), jnp.int32),
                pltpu.VMEM((CH, 16), jnp.float32),
                pltpu.VMEM((ZROWS, 16), jnp.float32),
                pltpu.VMEM_SHARED((ACC_ROWS, 16), jnp.float32),
                pltpu.SemaphoreType.DMA,
                pltpu.SemaphoreType.DMA,
                pltpu.SemaphoreType.DMA,
            ],
        )(_sc_deg_body)
    return _sc_cache["deg"](idx)


def _sc_deg_body(idx_h, out_h, si2, ones_v, zb, acc, ssem, isem, zsem):
    cid = lax.axis_index("c")
    sid = lax.axis_index("s")

    def zfill(i, carry):
        zb[i, pl.ds(0, 16)] = jnp.zeros((16,), jnp.float32)
        return carry

    lax.fori_loop(0, ZROWS, zfill, 0)

    def ofill(i, carry):
        ones_v[i, pl.ds(0, 16)] = jnp.full((16,), 1.0, jnp.float32)
        return carry

    lax.fori_loop(0, CH, ofill, 0)

    rowbase = sid * RPT
    row0 = cid * SROWS + sid * CPT

    pltpu.async_copy(
        idx_h.at[pl.ds(pl.multiple_of(row0, 8), GRP)], si2.at[0], isem)

    zhs = [
        pltpu.async_copy(zb, acc.at[pl.ds(rowbase + i * ZROWS, ZROWS)], zsem)
        for i in range(RPT // ZROWS)
    ]
    for h in zhs:
        h.wait()
    plsc.subcore_barrier()

    def scatter_drain():
        for _ in range(GRP):
            pltpu.make_async_copy(
                out_h.at[pl.ds(0, CH)], ones_v, ssem).wait()

    def group(g, p, q, first):
        pltpu.make_async_copy(
            idx_h.at[pl.ds(0, GRP)], si2.at[0], isem).wait()
        if not first:
            scatter_drain()
        off = pl.multiple_of(row0 + (g + 1) * GRP, 8)
        pltpu.async_copy(idx_h.at[pl.ds(off, GRP)], si2.at[q], isem)
        for j in range(GRP):
            pltpu.async_copy(ones_v, acc.at[si2.at[p, j]], ssem, add=True)

    group(0, 0, 1, True)

    def body(g, carry):
        p = lax.rem(g, 2)
        group(g, p, 1 - p, False)
        return carry

    lax.fori_loop(1, NGRP, body, 0)

    pltpu.make_async_copy(idx_h.at[pl.ds(0, GRP)], si2.at[0], isem).wait()
    scatter_drain()
    plsc.subcore_barrier()

    @pl.when(sid == 0)
    def _():
        pltpu.sync_copy(
            acc.at[pl.ds(0, NV)],
            out_h.at[pl.ds(pl.multiple_of(cid * NV, 8), NV)])


RB = 2000
NB = NV // RB


def _mlp_body(x_ref, w1_ref, b1_ref, w2_ref, b2_ref, o_ref):
    h = jnp.dot(x_ref[...], w1_ref[...], preferred_element_type=jnp.float32)
    h = jnp.maximum(h + b1_ref[...], 0.0)
    y = jnp.dot(h, w2_ref[...], preferred_element_type=jnp.float32)
    y = y + b2_ref[...]
    o_ref[0] = y[:, :HALF]
    o_ref[1] = y[:, HALF:]


def _mlp(x, w1, b1, w2, b2):
    return pl.pallas_call(
        _mlp_body,
        grid=(NB,),
        in_specs=[
            pl.BlockSpec((RB, DIN), lambda i: (i, 0)),
            pl.BlockSpec((DIN, DH), lambda i: (0, 0)),
            pl.BlockSpec((1, DH), lambda i: (0, 0)),
            pl.BlockSpec((DH, DH), lambda i: (0, 0)),
            pl.BlockSpec((1, DH), lambda i: (0, 0)),
        ],
        out_specs=pl.BlockSpec((2, RB, HALF), lambda i: (0, i, 0)),
        out_shape=jax.ShapeDtypeStruct((2, NV, HALF), jnp.float32),
    )(x, w1, b1.reshape(1, DH), w2, b2.reshape(1, DH))


def _update_body(split_out, x_ref, w_ref, b_ref, a_ref, d_ref, o_ref):
    x = jnp.concatenate([x_ref[0], x_ref[1]], axis=1)
    a = jnp.concatenate([a_ref[0], a_ref[1]], axis=1)
    inv = 1.0 / jnp.maximum(d_ref[:, 0:1], 1.0)
    y = jnp.dot(x, w_ref[...], preferred_element_type=jnp.float32)
    y = jnp.maximum(y + b_ref[...] + a * inv, 0.0)
    if split_out:
        o_ref[0] = y[:, :HALF]
        o_ref[1] = y[:, HALF:]
    else:
        o_ref[...] = y


def _update(x2, w, b, agg2, deg, split_out):
    if split_out:
        out_spec = pl.BlockSpec((2, RB, HALF), lambda i: (0, i, 0))
        out_shape = jax.ShapeDtypeStruct((2, NV, HALF), jnp.float32)
    else:
        out_spec = pl.BlockSpec((RB, DH), lambda i: (i, 0))
        out_shape = jax.ShapeDtypeStruct((NV, DH), jnp.float32)
    return pl.pallas_call(
        functools.partial(_update_body, split_out),
        grid=(NB,),
        in_specs=[
            pl.BlockSpec((2, RB, HALF), lambda i: (0, i, 0)),
            pl.BlockSpec((DH, DH), lambda i: (0, 0)),
            pl.BlockSpec((1, DH), lambda i: (0, 0)),
            pl.BlockSpec((2, RB, HALF), lambda i: (0, i, 0)),
            pl.BlockSpec((RB, 16), lambda i: (i, 0)),
        ],
        out_specs=out_spec,
        out_shape=out_shape,
    )(x2, w, b.reshape(1, DH), agg2, deg)


def _pad_chunks(idx16, pad_value):
    pad = jnp.full((NTILE, EPT_PAD - EPT), pad_value, jnp.int32)
    body = jnp.concatenate([idx16, pad], axis=1).reshape(SROWS, CH)
    tail = jnp.full((GRP, CH), pad_value, jnp.int32)
    return jnp.concatenate([body, tail], axis=0)


def kernel(var_feat, con_feat, edge_index, vW1, vb1, vW2, vb2, cW1, cb1, cW2,
           cb2, lvW, lvb, lcW, lcb):
    src16 = edge_index[0].reshape(NTILE, EPT)
    dst16 = edge_index[1].reshape(NTILE, EPT)

    g_from_c = _pad_chunks(src16, 0)
    g_from_v = _pad_chunks(dst16, 0)
    s_to_v = _pad_chunks(dst16, NV)
    s_to_c = _pad_chunks(src16, NV)
    deg_idx = jnp.concatenate([s_to_v[:SROWS], s_to_c], axis=0)

    degs = _sc_deg(deg_idx)
    deg_v = degs[:NV]
    deg_c = degs[NV:]

    v2 = _mlp(var_feat, vW1, vb1, vW2, vb2)
    c2 = _mlp(con_feat, cW1, cb1, cW2, cb2)

    for l in range(NLAYER):
        agg_cv = _sc_agg(c2.reshape(2 * NV, HALF), g_from_c, s_to_v)
        agg_cv2 = agg_cv.reshape(2, NV, HALF)
        if l < NLAYER - 1:
            v2 = _update(v2, lvW[l], lvb[l], agg_cv2, deg_v, True)
            agg_vc = _sc_agg(v2.reshape(2 * NV, HALF), g_from_v, s_to_c)
            c2 = _update(c2, lcW[l], lcb[l], agg_vc.reshape(2, NV, HALF),
                         deg_c, True)
        else:
            v_out = _update(v2, lvW[l], lvb[l], agg_cv2, deg_v, False)
    return v_out


# speedup vs baseline: 9.0203x; 1.0082x over previous
import functools

import jax
import jax.numpy as jnp
from jax import lax
from jax.experimental import pallas as pl
from jax.experimental.pallas import tpu as pltpu
from jax.experimental.pallas import tpu_sc as plsc

NV = 50000
NE = 800000
DIN = 32
DH = 64
HALF = 32
NLAYER = 3

NTILE = 16
CH = 128
GRP = 8
NBUF = 4
EPT = NE // NTILE
CPT = 392
EPT_PAD = CPT * CH
NGRP = CPT // GRP
SROWS = NTILE * CPT
RPT = NV // NTILE
ZROWS = 125
ACC_ROWS = NV + 8

_sc_cache = {}


def _sc_agg(table, gidx, sidx):
    if "agg" not in _sc_cache:
        mesh = plsc.VectorSubcoreMesh(core_axis_name="c", subcore_axis_name="s")
        _sc_cache["agg"] = functools.partial(
            pl.kernel,
            mesh=mesh,
            compiler_params=pltpu.CompilerParams(use_tc_tiling_on_sc=False),
            out_type=jax.ShapeDtypeStruct((2, NV, HALF), jnp.float32),
            scratch_types=[
                pltpu.VMEM((2, GRP, CH), jnp.int32),
                pltpu.VMEM((2, GRP, CH), jnp.int32),
                pltpu.VMEM((NBUF, CH, HALF), jnp.float32),
                pltpu.VMEM((ZROWS, HALF), jnp.float32),
                pltpu.VMEM_SHARED((ACC_ROWS, HALF), jnp.float32),
                pltpu.SemaphoreType.DMA,
                pltpu.SemaphoreType.DMA,
                pltpu.SemaphoreType.DMA,
                pltpu.SemaphoreType.DMA,
                pltpu.SemaphoreType.DMA,
            ],
        )(_sc_agg_body)
    return _sc_cache["agg"](table, gidx, sidx)


def _sc_agg_body(table_h, gidx_h, sidx_h, out_h, si2, gi2, rows, zb, acc,
                 sem0, sem1, sem2, sem3, isem):
    cid = lax.axis_index("c")
    sid = lax.axis_index("s")
    sems = (sem0, sem1, sem2, sem3)

    def zfill(i, carry):
        zb[i, pl.ds(0, 16)] = jnp.zeros((16,), jnp.float32)
        zb[i, pl.ds(16, 16)] = jnp.zeros((16,), jnp.float32)
        return carry

    lax.fori_loop(0, ZROWS, zfill, 0)

    rowbase = sid * RPT
    row0 = sid * CPT
    tbl = table_h.at[cid]

    off0 = pl.multiple_of(row0, 8)
    pltpu.async_copy(sidx_h.at[pl.ds(off0, GRP)], si2.at[0], isem)
    pltpu.async_copy(gidx_h.at[pl.ds(off0, GRP)], gi2.at[0], isem)

    zhs = [
        pltpu.async_copy(zb, acc.at[pl.ds(rowbase + i * ZROWS, ZROWS)], sem0)
        for i in range(RPT // ZROWS)
    ]
    for h in zhs:
        h.wait()
    plsc.subcore_barrier()

    def ring_wait(sem):
        pltpu.make_async_copy(tbl.at[pl.ds(0, CH)], rows.at[0], sem).wait()

    def idx_wait():
        pltpu.make_async_copy(
            sidx_h.at[pl.ds(0, GRP)], si2.at[0], isem).wait()
        pltpu.make_async_copy(
            gidx_h.at[pl.ds(0, GRP)], gi2.at[0], isem).wait()

    def group(g, p, q, first):
        idx_wait()
        for j in range(GRP):
            ka = j % NBUF
            if not (first and j < 2):
                kb = (j - 2) % NBUF
                ring_wait(sems[kb])
                if j < 2:
                    splane, srow = q, GRP - 2 + j
                else:
                    splane, srow = p, j - 2
                pltpu.async_copy(
                    rows.at[kb], acc.at[si2.at[splane, srow]], sems[kb],
                    add=True)
            if not (first and j < NBUF):
                ring_wait(sems[ka])
            pltpu.async_copy(tbl.at[gi2.at[p, j]], rows.at[ka], sems[ka])
            if j == 3:
                off = pl.multiple_of(row0 + (g + 1) * GRP, 8)
                pltpu.async_copy(sidx_h.at[pl.ds(off, GRP)], si2.at[q], isem)
                pltpu.async_copy(gidx_h.at[pl.ds(off, GRP)], gi2.at[q], isem)

    group(0, 0, 1, True)

    def body(g, carry):
        p = lax.rem(g, 2)
        group(g, p, 1 - p, False)
        return carry

    lax.fori_loop(1, NGRP, body, 0)

    for j in (GRP - 2, GRP - 1):
        k = j % NBUF
        ring_wait(sems[k])
        pltpu.async_copy(rows.at[k], acc.at[si2.at[0, j]], sems[k], add=True)
    for k in range(NBUF):
        ring_wait(sems[k])
    idx_wait()
    plsc.subcore_barrier()

    @pl.when(sid == 0)
    def _():
        pltpu.sync_copy(acc.at[pl.ds(0, NV)], out_h.at[cid])


def _sc_deg(idx):
    if "deg" not in _sc_cache:
        mesh = plsc.VectorSubcoreMesh(core_axis_name="c", subcore_axis_name="s")
        _sc_cache["deg"] = functools.partial(
            pl.kernel,
            mesh=mesh,
            compiler_params=pltpu.CompilerParams(use_tc_tiling_on_sc=False),
            out_type=jax.ShapeDtypeStruct((2 * NV, 16), jnp.float32),
            scratch_types=[
                pltpu.VMEM((2, GRP, CH), jnp.int32),
                pltpu.VMEM((CH, 16), jnp.float32),
                pltpu.VMEM((ZROWS, 16), jnp.float32),
                pltpu.VMEM_SHARED((ACC_ROWS, 16), jnp.float32),
                pltpu.SemaphoreType.DMA,
                pltpu.SemaphoreType.DMA,
                pltpu.SemaphoreType.DMA,
            ],
        )(_sc_deg_body)
    return _sc_cache["deg"](idx)


def _sc_deg_body(idx_h, out_h, si2, ones_v, zb, acc, ssem, isem, zsem):
    cid = lax.axis_index("c")
    sid = lax.axis_index("s")

    def zfill(i, carry):
        zb[i, pl.ds(0, 16)] = jnp.zeros((16,), jnp.float32)
        return carry

    lax.fori_loop(0, ZROWS, zfill, 0)

    def ofill(i, carry):
        ones_v[i, pl.ds(0, 16)] = jnp.full((16,), 1.0, jnp.float32)
        return carry

    lax.fori_loop(0, CH, ofill, 0)

    rowbase = sid * RPT
    row0 = cid * SROWS + sid * CPT

    pltpu.async_copy(
        idx_h.at[pl.ds(pl.multiple_of(row0, 8), GRP)], si2.at[0], isem)

    zhs = [
        pltpu.async_copy(zb, acc.at[pl.ds(rowbase + i * ZROWS, ZROWS)], zsem)
        for i in range(RPT // ZROWS)
    ]
    for h in zhs:
        h.wait()
    plsc.subcore_barrier()

    def scatter_drain():
        for _ in range(GRP):
            pltpu.make_async_copy(
                out_h.at[pl.ds(0, CH)], ones_v, ssem).wait()

    def group(g, p, q, first):
        pltpu.make_async_copy(
            idx_h.at[pl.ds(0, GRP)], si2.at[0], isem).wait()
        if not first:
            scatter_drain()
        off = pl.multiple_of(row0 + (g + 1) * GRP, 8)
        pltpu.async_copy(idx_h.at[pl.ds(off, GRP)], si2.at[q], isem)
        for j in range(GRP):
            pltpu.async_copy(ones_v, acc.at[si2.at[p, j]], ssem, add=True)

    group(0, 0, 1, True)

    def body(g, carry):
        p = lax.rem(g, 2)
        group(g, p, 1 - p, False)
        return carry

    lax.fori_loop(1, NGRP, body, 0)

    pltpu.make_async_copy(idx_h.at[pl.ds(0, GRP)], si2.at[0], isem).wait()
    scatter_drain()
    plsc.subcore_barrier()

    @pl.when(sid == 0)
    def _():
        pltpu.sync_copy(
            acc.at[pl.ds(0, NV)],
            out_h.at[pl.ds(pl.multiple_of(cid * NV, 8), NV)])


RB = 2000
NB = NV // RB


def _mlp_body(x_ref, w1_ref, b1_ref, w2_ref, b2_ref, o_ref):
    h = jnp.dot(x_ref[...], w1_ref[...], preferred_element_type=jnp.float32)
    h = jnp.maximum(h + b1_ref[...], 0.0)
    y = jnp.dot(h, w2_ref[...], preferred_element_type=jnp.float32)
    y = y + b2_ref[...]
    o_ref[0] = y[:, :HALF]
    o_ref[1] = y[:, HALF:]


def _mlp(x, w1, b1, w2, b2):
    return pl.pallas_call(
        _mlp_body,
        grid=(NB,),
        in_specs=[
            pl.BlockSpec((RB, DIN), lambda i: (i, 0)),
            pl.BlockSpec((DIN, DH), lambda i: (0, 0)),
            pl.BlockSpec((1, DH), lambda i: (0, 0)),
            pl.BlockSpec((DH, DH), lambda i: (0, 0)),
            pl.BlockSpec((1, DH), lambda i: (0, 0)),
        ],
        out_specs=pl.BlockSpec((2, RB, HALF), lambda i: (0, i, 0)),
        out_shape=jax.ShapeDtypeStruct((2, NV, HALF), jnp.float32),
    )(x, w1, b1.reshape(1, DH), w2, b2.reshape(1, DH))


def _premm_body(x_ref, w_ref, b_ref, o_ref):
    x = jnp.concatenate([x_ref[0], x_ref[1]], axis=1)
    y = jnp.dot(x, w_ref[...], preferred_element_type=jnp.float32)
    o_ref[...] = y + b_ref[...]


def _premm(x2, w, b):
    return pl.pallas_call(
        _premm_body,
        grid=(NB,),
        in_specs=[
            pl.BlockSpec((2, RB, HALF), lambda i: (0, i, 0)),
            pl.BlockSpec((DH, DH), lambda i: (0, 0)),
            pl.BlockSpec((1, DH), lambda i: (0, 0)),
        ],
        out_specs=pl.BlockSpec((RB, DH), lambda i: (i, 0)),
        out_shape=jax.ShapeDtypeStruct((NV, DH), jnp.float32),
    )(x2, w, b.reshape(1, DH))


def _post_body(split_out, z_ref, a_ref, d_ref, o_ref):
    a = jnp.concatenate([a_ref[0], a_ref[1]], axis=1)
    inv = 1.0 / jnp.maximum(d_ref[:, 0:1], 1.0)
    y = jnp.maximum(z_ref[...] + a * inv, 0.0)
    if split_out:
        o_ref[0] = y[:, :HALF]
        o_ref[1] = y[:, HALF:]
    else:
        o_ref[...] = y


def _post(z, agg2, degs, side, split_out):
    if split_out:
        out_spec = pl.BlockSpec((2, RB, HALF), lambda i: (0, i, 0))
        out_shape = jax.ShapeDtypeStruct((2, NV, HALF), jnp.float32)
    else:
        out_spec = pl.BlockSpec((RB, DH), lambda i: (i, 0))
        out_shape = jax.ShapeDtypeStruct((NV, DH), jnp.float32)
    return pl.pallas_call(
        functools.partial(_post_body, split_out),
        grid=(NB,),
        in_specs=[
            pl.BlockSpec((RB, DH), lambda i: (i, 0)),
            pl.BlockSpec((2, RB, HALF), lambda i: (0, i, 0)),
            pl.BlockSpec((RB, 16), lambda i, side=side: (side * NB + i, 0)),
        ],
        out_specs=out_spec,
        out_shape=out_shape,
    )(z, agg2, degs)


def _pad_chunks(idx16, pad_value):
    pad = jnp.full((NTILE, EPT_PAD - EPT), pad_value, jnp.int32)
    body = jnp.concatenate([idx16, pad], axis=1).reshape(SROWS, CH)
    tail = jnp.full((GRP, CH), pad_value, jnp.int32)
    return jnp.concatenate([body, tail], axis=0)


def kernel(var_feat, con_feat, edge_index, vW1, vb1, vW2, vb2, cW1, cb1, cW2,
           cb2, lvW, lvb, lcW, lcb):
    src16 = edge_index[0].reshape(NTILE, EPT)
    dst16 = edge_index[1].reshape(NTILE, EPT)

    g_from_c = _pad_chunks(src16, 0)
    g_from_v = _pad_chunks(dst16, 0)
    s_to_v = _pad_chunks(dst16, NV)
    s_to_c = _pad_chunks(src16, NV)
    deg_idx = jnp.concatenate([s_to_v[:SROWS], s_to_c], axis=0)

    degs = _sc_deg(deg_idx)

    v2 = _mlp(var_feat, vW1, vb1, vW2, vb2)
    c2 = _mlp(con_feat, cW1, cb1, cW2, cb2)

    for l in range(NLAYER):
        last = l == NLAYER - 1
        zv = _premm(v2, lvW[l], lvb[l])
        agg_cv = _sc_agg(c2, g_from_c, s_to_v)
        v2 = _post(zv, agg_cv, degs, 0, not last)
        if not last:
            zc = _premm(c2, lcW[l], lcb[l])
            agg_vc = _sc_agg(v2, g_from_v, s_to_c)
            c2 = _post(zc, agg_vc, degs, 1, True)
    return v2


# speedup vs baseline: 11.4783x; 1.2725x over previous
import functools

import jax
import jax.numpy as jnp
from jax import lax
from jax.experimental import pallas as pl
from jax.experimental.pallas import tpu as pltpu
from jax.experimental.pallas import tpu_sc as plsc

NV = 50000
NE = 800000
DIN = 32
DH = 64
HALF = 32
NLAYER = 3

NTILE = 16
CH = 128
GRP = 8
NBUF = 4
EPT = NE // NTILE
CPT = 392
EPT_PAD = CPT * CH
NGRP = CPT // GRP
SROWS = NTILE * CPT
RPT = NV // NTILE
ZROWS = 125
ACC_ROWS = NV + 8

_sc_cache = {}


def _sc_agg(table, gidx, sidx):
    if "agg" not in _sc_cache:
        mesh = plsc.VectorSubcoreMesh(core_axis_name="c", subcore_axis_name="s")
        _sc_cache["agg"] = functools.partial(
            pl.kernel,
            mesh=mesh,
            compiler_params=pltpu.CompilerParams(use_tc_tiling_on_sc=False),
            out_type=jax.ShapeDtypeStruct((2, NV, HALF), jnp.float32),
            scratch_types=[
                pltpu.VMEM((2, GRP, CH), jnp.int32),
                pltpu.VMEM((2, GRP, CH), jnp.int32),
                pltpu.VMEM((NBUF, CH, HALF), jnp.float32),
                pltpu.VMEM((ZROWS, HALF), jnp.float32),
                pltpu.VMEM_SHARED((ACC_ROWS, HALF), jnp.float32),
                pltpu.SemaphoreType.DMA,
                pltpu.SemaphoreType.DMA,
                pltpu.SemaphoreType.DMA,
                pltpu.SemaphoreType.DMA,
                pltpu.SemaphoreType.DMA,
            ],
        )(_sc_agg_body)
    return _sc_cache["agg"](table, gidx, sidx)


def _sc_agg_body(table_h, gidx_h, sidx_h, out_h, si2, gi2, rows, zb, acc,
                 sem0, sem1, sem2, sem3, isem):
    cid = lax.axis_index("c")
    sid = lax.axis_index("s")
    sems = (sem0, sem1, sem2, sem3)

    def zfill(i, carry):
        zb[i, pl.ds(0, 16)] = jnp.zeros((16,), jnp.float32)
        zb[i, pl.ds(16, 16)] = jnp.zeros((16,), jnp.float32)
        return carry

    lax.fori_loop(0, ZROWS, zfill, 0)

    rowbase = sid * RPT
    row0 = sid * CPT
    tbl = table_h.at[cid]

    off0 = pl.multiple_of(row0, 8)
    pltpu.async_copy(sidx_h.at[pl.ds(off0, GRP)], si2.at[0], isem)
    pltpu.async_copy(gidx_h.at[pl.ds(off0, GRP)], gi2.at[0], isem)

    zhs = [
        pltpu.async_copy(zb, acc.at[pl.ds(rowbase + i * ZROWS, ZROWS)], sem0)
        for i in range(RPT // ZROWS)
    ]
    for h in zhs:
        h.wait()
    plsc.subcore_barrier()

    def ring_wait(sem):
        pltpu.make_async_copy(tbl.at[pl.ds(0, CH)], rows.at[0], sem).wait()

    def idx_wait():
        pltpu.make_async_copy(
            sidx_h.at[pl.ds(0, GRP)], si2.at[0], isem).wait()
        pltpu.make_async_copy(
            gidx_h.at[pl.ds(0, GRP)], gi2.at[0], isem).wait()

    def group(g, p, q, first):
        idx_wait()
        for j in range(GRP):
            ka = j % NBUF
            if not (first and j < 2):
                kb = (j - 2) % NBUF
                ring_wait(sems[kb])
                if j < 2:
                    splane, srow = q, GRP - 2 + j
                else:
                    splane, srow = p, j - 2
                pltpu.async_copy(
                    rows.at[kb], acc.at[si2.at[splane, srow]], sems[kb],
                    add=True)
            if not (first and j < NBUF):
                ring_wait(sems[ka])
            pltpu.async_copy(tbl.at[gi2.at[p, j]], rows.at[ka], sems[ka])
            if j == 3:
                off = pl.multiple_of(row0 + (g + 1) * GRP, 8)
                pltpu.async_copy(sidx_h.at[pl.ds(off, GRP)], si2.at[q], isem)
                pltpu.async_copy(gidx_h.at[pl.ds(off, GRP)], gi2.at[q], isem)

    group(0, 0, 1, True)

    def body(g, carry):
        p = lax.rem(g, 2)
        group(g, p, 1 - p, False)
        return carry

    lax.fori_loop(1, NGRP, body, 0)

    for j in (GRP - 2, GRP - 1):
        k = j % NBUF
        ring_wait(sems[k])
        pltpu.async_copy(rows.at[k], acc.at[si2.at[0, j]], sems[k], add=True)
    for k in range(NBUF):
        ring_wait(sems[k])
    idx_wait()
    plsc.subcore_barrier()

    @pl.when(sid == 0)
    def _():
        pltpu.sync_copy(acc.at[pl.ds(0, NV)], out_h.at[cid])


def _sc_deg(idx):
    if "deg" not in _sc_cache:
        mesh = plsc.VectorSubcoreMesh(core_axis_name="c", subcore_axis_name="s")
        _sc_cache["deg"] = functools.partial(
            pl.kernel,
            mesh=mesh,
            compiler_params=pltpu.CompilerParams(use_tc_tiling_on_sc=False),
            out_type=jax.ShapeDtypeStruct((2, NV, HALF), jnp.float32),
            scratch_types=[
                pltpu.VMEM((2, GRP, CH), jnp.int32),
                pltpu.VMEM((CH, HALF), jnp.float32),
                pltpu.VMEM((ZROWS, HALF), jnp.float32),
                pltpu.VMEM_SHARED((ACC_ROWS, HALF), jnp.float32),
                pltpu.SemaphoreType.DMA,
                pltpu.SemaphoreType.DMA,
                pltpu.SemaphoreType.DMA,
            ],
        )(_sc_deg_body)
    return _sc_cache["deg"](idx)


def _sc_deg_body(idx_h, out_h, si2, ones_v, zb, acc, ssem, isem, zsem):
    cid = lax.axis_index("c")
    sid = lax.axis_index("s")

    def zfill(i, carry):
        zb[i, pl.ds(0, 16)] = jnp.zeros((16,), jnp.float32)
        zb[i, pl.ds(16, 16)] = jnp.zeros((16,), jnp.float32)
        return carry

    lax.fori_loop(0, ZROWS, zfill, 0)

    def ofill(i, carry):
        ones_v[i, pl.ds(0, 16)] = jnp.full((16,), 1.0, jnp.float32)
        ones_v[i, pl.ds(16, 16)] = jnp.full((16,), 1.0, jnp.float32)
        return carry

    lax.fori_loop(0, CH, ofill, 0)

    rowbase = sid * RPT
    row0 = cid * SROWS + sid * CPT

    pltpu.async_copy(
        idx_h.at[pl.ds(pl.multiple_of(row0, 8), GRP)], si2.at[0], isem)

    zhs = [
        pltpu.async_copy(zb, acc.at[pl.ds(rowbase + i * ZROWS, ZROWS)], zsem)
        for i in range(RPT // ZROWS)
    ]
    for h in zhs:
        h.wait()
    plsc.subcore_barrier()

    def scatter_drain():
        for _ in range(GRP):
            pltpu.make_async_copy(
                acc.at[pl.ds(0, CH)], ones_v, ssem).wait()

    def group(g, p, q, first):
        pltpu.make_async_copy(
            idx_h.at[pl.ds(0, GRP)], si2.at[0], isem).wait()
        if not first:
            scatter_drain()
        off = pl.multiple_of(row0 + (g + 1) * GRP, 8)
        pltpu.async_copy(idx_h.at[pl.ds(off, GRP)], si2.at[q], isem)
        for j in range(GRP):
            pltpu.async_copy(ones_v, acc.at[si2.at[p, j]], ssem, add=True)

    group(0, 0, 1, True)

    def body(g, carry):
        p = lax.rem(g, 2)
        group(g, p, 1 - p, False)
        return carry

    lax.fori_loop(1, NGRP, body, 0)

    pltpu.make_async_copy(idx_h.at[pl.ds(0, GRP)], si2.at[0], isem).wait()
    scatter_drain()
    plsc.subcore_barrier()

    @pl.when(sid == 0)
    def _():
        pltpu.sync_copy(acc.at[pl.ds(0, NV)], out_h.at[cid])


RB = 2000
NB = NV // RB
RB4 = RB // 4
FSHAPE = (2, NB, RB4, 128)


def _blockdiag(w):
    eye = jnp.eye(4, dtype=w.dtype)
    return jnp.einsum('gG,ij->giGj', eye, w).reshape(
        4 * w.shape[0], 4 * w.shape[1])


def _mlp_weights(w1, b1, w2, b2):
    w441 = _blockdiag(w1)
    b1p = jnp.tile(b1, 4).reshape(1, 4 * DH)
    eye = jnp.eye(4, dtype=w2.dtype)
    w2r = w2.reshape(DH, 2, HALF)
    w442 = jnp.einsum('gG,mhf->hgmGf', eye, w2r).reshape(2, 4 * DH, 128)
    b2p = jnp.tile(b2.reshape(2, 1, HALF), (1, 4, 1)).reshape(2, 128)
    return w441, b1p, w442, b2p


def _layer_weights(w, b):
    eye = jnp.eye(4, dtype=w.dtype)
    wr = w.reshape(2, HALF, 2, HALF)
    w44 = jnp.einsum('gG,pihf->hpgiGf', eye, wr).reshape(2, 2 * 128, 128)
    b128 = jnp.tile(b.reshape(2, 1, HALF), (1, 4, 1)).reshape(2, 128)
    return w44, b128


def _mlp_body(x_ref, w1_ref, b1_ref, w2_ref, b2_ref, o_ref):
    h = jnp.dot(x_ref[0], w1_ref[...], preferred_element_type=jnp.float32)
    h = jnp.maximum(h + b1_ref[...], 0.0)
    for s in range(2):
        y = jnp.dot(h, w2_ref[s], preferred_element_type=jnp.float32)
        o_ref[s, 0] = y + b2_ref[s:s + 1]


def _mlp(xp, w1, b1, w2, b2):
    w441, b1p, w442, b2p = _mlp_weights(w1, b1, w2, b2)
    return pl.pallas_call(
        _mlp_body,
        grid=(NB,),
        in_specs=[
            pl.BlockSpec((1, RB4, 128), lambda i: (i, 0, 0)),
            pl.BlockSpec((128, 4 * DH), lambda i: (0, 0)),
            pl.BlockSpec((1, 4 * DH), lambda i: (0, 0)),
            pl.BlockSpec((2, 4 * DH, 128), lambda i: (0, 0, 0)),
            pl.BlockSpec((2, 128), lambda i: (0, 0)),
        ],
        out_specs=pl.BlockSpec((2, 1, RB4, 128), lambda i: (0, i, 0, 0)),
        out_shape=jax.ShapeDtypeStruct(FSHAPE, jnp.float32),
    )(xp, w441, b1p, w442, b2p)


def _premm_body(x_ref, w_ref, b_ref, o_ref):
    p2 = jnp.concatenate([x_ref[0, 0], x_ref[1, 0]], axis=1)
    for s in range(2):
        y = jnp.dot(p2, w_ref[s], preferred_element_type=jnp.float32)
        o_ref[s, 0] = y + b_ref[s:s + 1]


def _premm(x2, w, b):
    w44, b128 = _layer_weights(w, b)
    return pl.pallas_call(
        _premm_body,
        grid=(NB,),
        in_specs=[
            pl.BlockSpec((2, 1, RB4, 128), lambda i: (0, i, 0, 0)),
            pl.BlockSpec((2, 256, 128), lambda i: (0, 0, 0)),
            pl.BlockSpec((2, 128), lambda i: (0, 0)),
        ],
        out_specs=pl.BlockSpec((2, 1, RB4, 128), lambda i: (0, i, 0, 0)),
        out_shape=jax.ShapeDtypeStruct(FSHAPE, jnp.float32),
    )(x2, w44, b128)


def _post_body(z_ref, a_ref, d_ref, o_ref):
    inv = 1.0 / jnp.maximum(d_ref[0, 0], 1.0)
    for s in range(2):
        o_ref[s, 0] = jnp.maximum(z_ref[s, 0] + a_ref[s, 0] * inv, 0.0)


def _post(z, agg2, degs, side):
    return pl.pallas_call(
        _post_body,
        grid=(NB,),
        in_specs=[
            pl.BlockSpec((2, 1, RB4, 128), lambda i: (0, i, 0, 0)),
            pl.BlockSpec((2, 1, RB4, 128), lambda i: (0, i, 0, 0)),
            pl.BlockSpec((1, 1, RB4, 128),
                         lambda i, side=side: (side, i, 0, 0)),
        ],
        out_specs=pl.BlockSpec((2, 1, RB4, 128), lambda i: (0, i, 0, 0)),
        out_shape=jax.ShapeDtypeStruct(FSHAPE, jnp.float32),
    )(z, agg2, degs)


def _pad_chunks(idx16, pad_value):
    pad = jnp.full((NTILE, EPT_PAD - EPT), pad_value, jnp.int32)
    body = jnp.concatenate([idx16, pad], axis=1).reshape(SROWS, CH)
    tail = jnp.full((GRP, CH), pad_value, jnp.int32)
    return jnp.concatenate([body, tail], axis=0)


def kernel(var_feat, con_feat, edge_index, vW1, vb1, vW2, vb2, cW1, cb1, cW2,
           cb2, lvW, lvb, lcW, lcb):
    src16 = edge_index[0].reshape(NTILE, EPT)
    dst16 = edge_index[1].reshape(NTILE, EPT)

    g_from_c = _pad_chunks(src16, 0)
    g_from_v = _pad_chunks(dst16, 0)
    s_to_v = _pad_chunks(dst16, NV)
    s_to_c = _pad_chunks(src16, NV)
    deg_idx = jnp.concatenate([s_to_v[:SROWS], s_to_c], axis=0)

    degs = _sc_deg(deg_idx).reshape(FSHAPE)

    def sc_view(x2):
        return x2.reshape(2, NV, HALF)

    def tc_view(x2):
        return x2.reshape(FSHAPE)

    xvp = var_feat.reshape(NB, RB4, 128)
    xcp = con_feat.reshape(NB, RB4, 128)
    v2 = _mlp(xvp, vW1, vb1, vW2, vb2)
    c2 = _mlp(xcp, cW1, cb1, cW2, cb2)

    for l in range(NLAYER):
        last = l == NLAYER - 1
        zv = _premm(v2, lvW[l], lvb[l])
        agg_cv = _sc_agg(sc_view(c2), g_from_c, s_to_v)
        v2 = _post(zv, tc_view(agg_cv), degs, 0)
        if not last:
            zc = _premm(c2, lcW[l], lcb[l])
            agg_vc = _sc_agg(sc_view(v2), g_from_v, s_to_c)
            c2 = _post(zc, tc_view(agg_vc), degs, 1)
    return jnp.swapaxes(v2.reshape(2, NV, HALF), 0, 1).reshape(NV, DH)


# speedup vs baseline: 11.9229x; 1.0387x over previous
import functools

import jax
import jax.numpy as jnp
from jax import lax
from jax.experimental import pallas as pl
from jax.experimental.pallas import tpu as pltpu
from jax.experimental.pallas import tpu_sc as plsc

NV = 50000
NE = 800000
DIN = 32
DH = 64
HALF = 32
NLAYER = 3

NTILE = 16
CH = 128
GRP = 8
NBUF = 4
EPT = NE // NTILE
CPT = 392
EPT_PAD = CPT * CH
NGRP = CPT // GRP
SROWS = NTILE * CPT
NVP = 50176
RPT = NVP // NTILE
ZROWS = 112
ACC_ROWS = NVP

_sc_cache = {}


def _sc_agg(table, gidx, sidx):
    if "agg" not in _sc_cache:
        mesh = plsc.VectorSubcoreMesh(core_axis_name="c", subcore_axis_name="s")
        _sc_cache["agg"] = functools.partial(
            pl.kernel,
            mesh=mesh,
            compiler_params=pltpu.CompilerParams(use_tc_tiling_on_sc=False),
            out_type=jax.ShapeDtypeStruct((2, NVP, HALF), jnp.float32),
            scratch_types=[
                pltpu.VMEM((2, GRP, CH), jnp.int32),
                pltpu.VMEM((2, GRP, CH), jnp.int32),
                pltpu.VMEM((NBUF, CH, HALF), jnp.float32),
                pltpu.VMEM((ZROWS, HALF), jnp.float32),
                pltpu.VMEM_SHARED((ACC_ROWS, HALF), jnp.float32),
                pltpu.SemaphoreType.DMA,
                pltpu.SemaphoreType.DMA,
                pltpu.SemaphoreType.DMA,
                pltpu.SemaphoreType.DMA,
                pltpu.SemaphoreType.DMA,
            ],
        )(_sc_agg_body)
    return _sc_cache["agg"](table, gidx, sidx)


def _sc_agg_body(table_h, gidx_h, sidx_h, out_h, si2, gi2, rows, zb, acc,
                 sem0, sem1, sem2, sem3, isem):
    cid = lax.axis_index("c")
    sid = lax.axis_index("s")
    sems = (sem0, sem1, sem2, sem3)

    def zfill(i, carry):
        zb[i, pl.ds(0, 16)] = jnp.zeros((16,), jnp.float32)
        zb[i, pl.ds(16, 16)] = jnp.zeros((16,), jnp.float32)
        return carry

    lax.fori_loop(0, ZROWS, zfill, 0)

    rowbase = sid * RPT
    row0 = sid * CPT
    tbl = table_h.at[cid]

    off0 = pl.multiple_of(row0, 8)
    pltpu.async_copy(sidx_h.at[pl.ds(off0, GRP)], si2.at[0], isem)
    pltpu.async_copy(gidx_h.at[pl.ds(off0, GRP)], gi2.at[0], isem)

    zhs = [
        pltpu.async_copy(zb, acc.at[pl.ds(rowbase + i * ZROWS, ZROWS)], sem0)
        for i in range(RPT // ZROWS)
    ]
    for h in zhs:
        h.wait()
    plsc.subcore_barrier()

    def ring_wait(sem):
        pltpu.make_async_copy(tbl.at[pl.ds(0, CH)], rows.at[0], sem).wait()

    def idx_wait():
        pltpu.make_async_copy(
            sidx_h.at[pl.ds(0, GRP)], si2.at[0], isem).wait()
        pltpu.make_async_copy(
            gidx_h.at[pl.ds(0, GRP)], gi2.at[0], isem).wait()

    def group(g, p, q, first):
        idx_wait()
        for j in range(GRP):
            ka = j % NBUF
            if not (first and j < 2):
                kb = (j - 2) % NBUF
                ring_wait(sems[kb])
                if j < 2:
                    splane, srow = q, GRP - 2 + j
                else:
                    splane, srow = p, j - 2
                pltpu.async_copy(
                    rows.at[kb], acc.at[si2.at[splane, srow]], sems[kb],
                    add=True)
            if not (first and j < NBUF):
                ring_wait(sems[ka])
            pltpu.async_copy(tbl.at[gi2.at[p, j]], rows.at[ka], sems[ka])
            if j == 3:
                off = pl.multiple_of(row0 + (g + 1) * GRP, 8)
                pltpu.async_copy(sidx_h.at[pl.ds(off, GRP)], si2.at[q], isem)
                pltpu.async_copy(gidx_h.at[pl.ds(off, GRP)], gi2.at[q], isem)

    group(0, 0, 1, True)

    def body(g, carry):
        p = lax.rem(g, 2)
        group(g, p, 1 - p, False)
        return carry

    lax.fori_loop(1, NGRP, body, 0)

    for j in (GRP - 2, GRP - 1):
        k = j % NBUF
        ring_wait(sems[k])
        pltpu.async_copy(rows.at[k], acc.at[si2.at[0, j]], sems[k], add=True)
    for k in range(NBUF):
        ring_wait(sems[k])
    idx_wait()
    plsc.subcore_barrier()

    @pl.when(sid == 0)
    def _():
        pltpu.sync_copy(acc.at[pl.ds(0, NVP)], out_h.at[cid])


def _sc_deg(idx):
    if "deg" not in _sc_cache:
        mesh = plsc.VectorSubcoreMesh(core_axis_name="c", subcore_axis_name="s")
        _sc_cache["deg"] = functools.partial(
            pl.kernel,
            mesh=mesh,
            compiler_params=pltpu.CompilerParams(use_tc_tiling_on_sc=False),
            out_type=jax.ShapeDtypeStruct((2, NVP, HALF), jnp.float32),
            scratch_types=[
                pltpu.VMEM((2, GRP, CH), jnp.int32),
                pltpu.VMEM((CH, HALF), jnp.float32),
                pltpu.VMEM((ZROWS, HALF), jnp.float32),
                pltpu.VMEM_SHARED((ACC_ROWS, HALF), jnp.float32),
                pltpu.SemaphoreType.DMA,
                pltpu.SemaphoreType.DMA,
                pltpu.SemaphoreType.DMA,
            ],
        )(_sc_deg_body)
    return _sc_cache["deg"](idx)


def _sc_deg_body(idx_h, out_h, si2, ones_v, zb, acc, ssem, isem, zsem):
    cid = lax.axis_index("c")
    sid = lax.axis_index("s")

    def zfill(i, carry):
        zb[i, pl.ds(0, 16)] = jnp.zeros((16,), jnp.float32)
        zb[i, pl.ds(16, 16)] = jnp.zeros((16,), jnp.float32)
        return carry

    lax.fori_loop(0, ZROWS, zfill, 0)

    def ofill(i, carry):
        ones_v[i, pl.ds(0, 16)] = jnp.full((16,), 1.0, jnp.float32)
        ones_v[i, pl.ds(16, 16)] = jnp.full((16,), 1.0, jnp.float32)
        return carry

    lax.fori_loop(0, CH, ofill, 0)

    rowbase = sid * RPT
    row0 = cid * SROWS + sid * CPT

    pltpu.async_copy(
        idx_h.at[pl.ds(pl.multiple_of(row0, 8), GRP)], si2.at[0], isem)

    zhs = [
        pltpu.async_copy(zb, acc.at[pl.ds(rowbase + i * ZROWS, ZROWS)], zsem)
        for i in range(RPT // ZROWS)
    ]
    for h in zhs:
        h.wait()
    plsc.subcore_barrier()

    def scatter_drain():
        for _ in range(GRP):
            pltpu.make_async_copy(
                acc.at[pl.ds(0, CH)], ones_v, ssem).wait()

    def group(g, p, q, first):
        pltpu.make_async_copy(
            idx_h.at[pl.ds(0, GRP)], si2.at[0], isem).wait()
        if not first:
            scatter_drain()
        off = pl.multiple_of(row0 + (g + 1) * GRP, 8)
        pltpu.async_copy(idx_h.at[pl.ds(off, GRP)], si2.at[q], isem)
        for j in range(GRP):
            pltpu.async_copy(ones_v, acc.at[si2.at[p, j]], ssem, add=True)

    group(0, 0, 1, True)

    def body(g, carry):
        p = lax.rem(g, 2)
        group(g, p, 1 - p, False)
        return carry

    lax.fori_loop(1, NGRP, body, 0)

    pltpu.make_async_copy(idx_h.at[pl.ds(0, GRP)], si2.at[0], isem).wait()
    scatter_drain()
    plsc.subcore_barrier()

    @pl.when(sid == 0)
    def _():
        pltpu.sync_copy(acc.at[pl.ds(0, NVP)], out_h.at[cid])


RB = 1792
NB = NVP // RB
RB4 = RB // 4
FSHAPE = (2, NB, RB4, 128)


def _blockdiag(w):
    eye = jnp.eye(4, dtype=w.dtype)
    return jnp.einsum('gG,ij->giGj', eye, w).reshape(
        4 * w.shape[0], 4 * w.shape[1])


def _mlp_weights(w1, b1, w2, b2):
    w441 = _blockdiag(w1)
    b1p = jnp.tile(b1, 4).reshape(1, 4 * DH)
    eye = jnp.eye(4, dtype=w2.dtype)
    w2r = w2.reshape(DH, 2, HALF)
    w442 = jnp.einsum('gG,mhf->hgmGf', eye, w2r).reshape(2, 4 * DH, 128)
    b2p = jnp.tile(b2.reshape(2, 1, HALF), (1, 4, 1)).reshape(2, 128)
    return w441, b1p, w442, b2p


def _layer_weights(w, b):
    eye = jnp.eye(4, dtype=w.dtype)
    wr = w.reshape(2, HALF, 2, HALF)
    w44 = jnp.einsum('gG,pihf->hpgiGf', eye, wr).reshape(2, 2 * 128, 128)
    b128 = jnp.tile(b.reshape(2, 1, HALF), (1, 4, 1)).reshape(2, 128)
    return w44, b128


def _mlp_body(x_ref, w1_ref, b1_ref, w2_ref, b2_ref, o_ref):
    h = jnp.dot(x_ref[0], w1_ref[...], preferred_element_type=jnp.float32)
    h = jnp.maximum(h + b1_ref[...], 0.0)
    for s in range(2):
        y = jnp.dot(h, w2_ref[s], preferred_element_type=jnp.float32)
        o_ref[s, 0] = y + b2_ref[s:s + 1]


def _mlp(xp, w1, b1, w2, b2):
    w441, b1p, w442, b2p = _mlp_weights(w1, b1, w2, b2)
    return pl.pallas_call(
        _mlp_body,
        grid=(NB,),
        in_specs=[
            pl.BlockSpec((1, RB4, 128), lambda i: (i, 0, 0)),
            pl.BlockSpec((128, 4 * DH), lambda i: (0, 0)),
            pl.BlockSpec((1, 4 * DH), lambda i: (0, 0)),
            pl.BlockSpec((2, 4 * DH, 128), lambda i: (0, 0, 0)),
            pl.BlockSpec((2, 128), lambda i: (0, 0)),
        ],
        out_specs=pl.BlockSpec((2, 1, RB4, 128), lambda i: (0, i, 0, 0)),
        out_shape=jax.ShapeDtypeStruct(FSHAPE, jnp.float32),
    )(xp, w441, b1p, w442, b2p)


def _premm_body(x_ref, w_ref, b_ref, o_ref):
    p2 = jnp.concatenate([x_ref[0, 0], x_ref[1, 0]], axis=1)
    for s in range(2):
        y = jnp.dot(p2, w_ref[s], preferred_element_type=jnp.float32)
        o_ref[s, 0] = y + b_ref[s:s + 1]


def _premm(x2, w, b):
    w44, b128 = _layer_weights(w, b)
    return pl.pallas_call(
        _premm_body,
        grid=(NB,),
        in_specs=[
            pl.BlockSpec((2, 1, RB4, 128), lambda i: (0, i, 0, 0)),
            pl.BlockSpec((2, 256, 128), lambda i: (0, 0, 0)),
            pl.BlockSpec((2, 128), lambda i: (0, 0)),
        ],
        out_specs=pl.BlockSpec((2, 1, RB4, 128), lambda i: (0, i, 0, 0)),
        out_shape=jax.ShapeDtypeStruct(FSHAPE, jnp.float32),
    )(x2, w44, b128)


def _post_body(z_ref, a_ref, d_ref, o_ref):
    inv = 1.0 / jnp.maximum(d_ref[0, 0], 1.0)
    for s in range(2):
        o_ref[s, 0] = jnp.maximum(z_ref[s, 0] + a_ref[s, 0] * inv, 0.0)


def _post(z, agg2, degs, side):
    return pl.pallas_call(
        _post_body,
        grid=(NB,),
        in_specs=[
            pl.BlockSpec((2, 1, RB4, 128), lambda i: (0, i, 0, 0)),
            pl.BlockSpec((2, 1, RB4, 128), lambda i: (0, i, 0, 0)),
            pl.BlockSpec((1, 1, RB4, 128),
                         lambda i, side=side: (side, i, 0, 0)),
        ],
        out_specs=pl.BlockSpec((2, 1, RB4, 128), lambda i: (0, i, 0, 0)),
        out_shape=jax.ShapeDtypeStruct(FSHAPE, jnp.float32),
    )(z, agg2, degs)


def _pad_chunks(idx16, pad_value):
    pad = jnp.full((NTILE, EPT_PAD - EPT), pad_value, jnp.int32)
    body = jnp.concatenate([idx16, pad], axis=1).reshape(SROWS, CH)
    tail = jnp.full((GRP, CH), pad_value, jnp.int32)
    return jnp.concatenate([body, tail], axis=0)


def kernel(var_feat, con_feat, edge_index, vW1, vb1, vW2, vb2, cW1, cb1, cW2,
           cb2, lvW, lvb, lcW, lcb):
    src16 = edge_index[0].reshape(NTILE, EPT)
    dst16 = edge_index[1].reshape(NTILE, EPT)

    g_from_c = _pad_chunks(src16, 0)
    g_from_v = _pad_chunks(dst16, 0)
    s_to_v = _pad_chunks(dst16, NV)
    s_to_c = _pad_chunks(src16, NV)
    deg_idx = jnp.concatenate([s_to_v[:SROWS], s_to_c], axis=0)

    degs = _sc_deg(deg_idx).reshape(FSHAPE)

    def sc_view(x2):
        return x2.reshape(2, NVP, HALF)

    def tc_view(x2):
        return x2.reshape(FSHAPE)

    pad = ((0, NVP - NV), (0, 0))
    xvp = jnp.pad(var_feat, pad).reshape(NB, RB4, 128)
    xcp = jnp.pad(con_feat, pad).reshape(NB, RB4, 128)
    v2 = _mlp(xvp, vW1, vb1, vW2, vb2)
    c2 = _mlp(xcp, cW1, cb1, cW2, cb2)

    for l in range(NLAYER):
        last = l == NLAYER - 1
        zv = _premm(v2, lvW[l], lvb[l])
        agg_cv = _sc_agg(sc_view(c2), g_from_c, s_to_v)
        v2 = _post(zv, tc_view(agg_cv), degs, 0)
        if not last:
            zc = _premm(c2, lcW[l], lcb[l])
            agg_vc = _sc_agg(sc_view(v2), g_from_v, s_to_c)
            c2 = _post(zc, tc_view(agg_vc), degs, 1)
    v2 = v2.reshape(2, NVP, HALF)[:, :NV]
    return jnp.swapaxes(v2, 0, 1).reshape(NV, DH)
